# baseline (speedup 1.0000x reference)
import jax
import jax.numpy as jnp
from jax import lax
from jax.experimental import pallas as pl
from jax.experimental.pallas import tpu as pltpu
from jax.experimental.pallas import tpu_sc as plsc

_N = 16384
_C = 1000
_NW = 32
_RPW = _N // _NW
_CH = 64
_NCH = _RPW // _CH
_BUF = _CH * _C


def _sc_body(x_hbm, o_hbm, buf, idx_v):
    cid = lax.axis_index("c")
    sid = lax.axis_index("s")
    wid = sid * 2 + cid
    base = wid * _RPW

    z = jnp.zeros((16,), jnp.float32)

    def zero_blk(i, _):
        for u in range(8):
            buf[pl.ds(i * 128 + u * 16, 16)] = z
        return _

    lax.fori_loop(0, _BUF // 128, zero_blk, None)

    ones = jnp.full((16,), 1.0, jnp.float32)
    lane = lax.iota(jnp.int32, 16)

    for k in range(_NCH):
        row_lo = base + k * _CH
        pltpu.sync_copy(x_hbm.at[pl.ds(row_lo, _CH)], idx_v)
        for g in range(_CH // 16):
            pos = (g * 16 + lane) * _C + idx_v[pl.ds(g * 16, 16)]
            plsc.store_scatter(buf, [pos], ones)
        pltpu.sync_copy(buf, o_hbm.at[pl.ds(row_lo * _C, _BUF)])
        for g in range(_CH // 16):
            pos = (g * 16 + lane) * _C + idx_v[pl.ds(g * 16, 16)]
            plsc.store_scatter(buf, [pos], z)


_mesh = plsc.VectorSubcoreMesh(core_axis_name="c", subcore_axis_name="s")

_sc_call = pl.kernel(
    _sc_body,
    out_type=jax.ShapeDtypeStruct((_N * _C,), jnp.float32),
    mesh=_mesh,
    scratch_types=[
        pltpu.VMEM((_BUF,), jnp.float32),
        pltpu.VMEM((_CH,), jnp.int32),
    ],
    compiler_params=pltpu.CompilerParams(needs_layout_passes=False),
)


def kernel(x):
    return _sc_call(x.astype(jnp.int32)).reshape(_N, _C)


# speedup vs baseline: 1.6279x; 1.6279x over previous
import jax
import jax.numpy as jnp
from jax import lax
from jax.experimental import pallas as pl
from jax.experimental.pallas import tpu as pltpu
from jax.experimental.pallas import tpu_sc as plsc

_N = 16384
_C = 1000
_NW = 32
_RPW = _N // _NW
_CH = 64
_NCH = _RPW // _CH


def _sc_body(x_hbm, o_hbm, buf, idx_v):
    cid = lax.axis_index("c")
    sid = lax.axis_index("s")
    wid = sid * 2 + cid
    base = wid * _RPW

    z = jnp.zeros((16,), jnp.float32)

    def zero_row(r, _):
        for c in range(_C // 16):
            buf[r, pl.ds(c * 16, 16)] = z
        buf[r, pl.ds(_C - 16, 16)] = z
        return _

    lax.fori_loop(0, _CH, zero_row, None)

    pltpu.sync_copy(x_hbm.at[pl.ds(base, _RPW)], idx_v)

    ones = jnp.full((16,), 1.0, jnp.float32)
    lane = lax.iota(jnp.int32, 16)

    for k in range(_NCH):
        for g in range(_CH // 16):
            rvec = g * 16 + lane
            xv = idx_v[pl.ds(k * _CH + g * 16, 16)]
            plsc.store_scatter(buf, [rvec, xv], ones)
        pltpu.sync_copy(buf, o_hbm.at[pl.ds(base + k * _CH, _CH), :])
        for g in range(_CH // 16):
            rvec = g * 16 + lane
            xv = idx_v[pl.ds(k * _CH + g * 16, 16)]
            plsc.store_scatter(buf, [rvec, xv], z)


_mesh = plsc.VectorSubcoreMesh(core_axis_name="c", subcore_axis_name="s")

_sc_call = pl.kernel(
    _sc_body,
    out_type=jax.ShapeDtypeStruct((_N, _C), jnp.float32),
    mesh=_mesh,
    scratch_types=[
        pltpu.VMEM((_CH, _C), jnp.float32),
        pltpu.VMEM((_RPW,), jnp.int32),
    ],
    compiler_params=pltpu.CompilerParams(needs_layout_passes=False),
)


def kernel(x):
    return _sc_call(x.astype(jnp.int32))


# speedup vs baseline: 1.6408x; 1.0079x over previous
import jax
import jax.numpy as jnp
from jax import lax
from jax.experimental import pallas as pl
from jax.experimental.pallas import tpu as pltpu
from jax.experimental.pallas import tpu_sc as plsc

_N = 16384
_C = 1000
_NW = 32
_RPW = _N // _NW
_CH = 32
_NCH = _RPW // _CH
_NBUF = 2


def _sc_body(x_hbm, o_hbm, buf, idx_v, sems):
    cid = lax.axis_index("c")
    sid = lax.axis_index("s")
    wid = sid * 2 + cid
    base = wid * _RPW

    z = jnp.zeros((16,), jnp.float32)

    def zero_row(r, _):
        for b in range(_NBUF):
            for c in range(_C // 16):
                buf[b, r, pl.ds(c * 16, 16)] = z
            buf[b, r, pl.ds(_C - 16, 16)] = z
        return _

    lax.fori_loop(0, _CH, zero_row, None)

    pltpu.sync_copy(x_hbm.at[pl.ds(base, _RPW)], idx_v)

    ones = jnp.full((16,), 1.0, jnp.float32)
    lane = lax.iota(jnp.int32, 16)

    def scatter(k, val):
        s = k % _NBUF
        for g in range(_CH // 16):
            rvec = g * 16 + lane
            xv = idx_v[pl.ds(k * _CH + g * 16, 16)]
            plsc.store_scatter(buf.at[s], [rvec, xv], val)

    def copy(k):
        return pltpu.make_async_copy(
            buf.at[k % _NBUF],
            o_hbm.at[pl.ds(base + k * _CH, _CH), :],
            sems.at[k % _NBUF],
        )

    for k in range(_NCH):
        if k >= _NBUF:
            copy(k - _NBUF).wait()
            scatter(k - _NBUF, z)
        scatter(k, ones)
        copy(k).start()
    for k in range(_NCH - _NBUF, _NCH):
        copy(k).wait()


_mesh = plsc.VectorSubcoreMesh(core_axis_name="c", subcore_axis_name="s")

_sc_call = pl.kernel(
    _sc_body,
    out_type=jax.ShapeDtypeStruct((_N, _C), jnp.float32),
    mesh=_mesh,
    scratch_types=[
        pltpu.VMEM((_NBUF, _CH, _C), jnp.float32),
        pltpu.VMEM((_RPW,), jnp.int32),
        pltpu.SemaphoreType.DMA((_NBUF,)),
    ],
    compiler_params=pltpu.CompilerParams(needs_layout_passes=False),
)


def kernel(x):
    return _sc_call(x.astype(jnp.int32))


# speedup vs baseline: 1.7893x; 1.0905x over previous
import jax
import jax.numpy as jnp
from jax.experimental import pallas as pl
from jax.experimental.pallas import tpu as pltpu

_N = 16384
_C = 1000
_CP = 1024
_ROWS = 2048


def _onehot_block(x_ref, o_ref):
    xv = x_ref[...]
    col = jax.lax.broadcasted_iota(jnp.int32, (_ROWS, _CP), 1)
    o_ref[...] = (xv == col).astype(jnp.float32)


def kernel(x):
    x2 = x.reshape(_N, 1).astype(jnp.int32)
    padded = pl.pallas_call(
        _onehot_block,
        grid=(_N // _ROWS,),
        in_specs=[pl.BlockSpec((_ROWS, 1), lambda i: (i, 0))],
        out_specs=pl.BlockSpec((_ROWS, _CP), lambda i: (i, 0)),
        out_shape=jax.ShapeDtypeStruct((_N, _CP), jnp.float32),
        compiler_params=pltpu.CompilerParams(
            dimension_semantics=("arbitrary",),
        ),
    )(x2)
    return jax.lax.slice(padded, (0, 0), (_N, _C))


# speedup vs baseline: 2.0620x; 1.1524x over previous
import jax
import jax.numpy as jnp
from jax.experimental import pallas as pl
from jax.experimental.pallas import tpu as pltpu

_N = 16384
_C = 1000
_ROWS = 1024


def _onehot_block(x_ref, o_ref):
    xv = x_ref[0].reshape(_ROWS, 1)
    col = jax.lax.broadcasted_iota(jnp.int32, (_ROWS, _C), 1)
    o_ref[...] = (xv == col).astype(jnp.float32)


def kernel(x):
    x3 = x.reshape(_N // _ROWS, 1, _ROWS).astype(jnp.int32)
    return pl.pallas_call(
        _onehot_block,
        grid=(_N // _ROWS,),
        in_specs=[pl.BlockSpec((1, 1, _ROWS), lambda i: (i, 0, 0))],
        out_specs=pl.BlockSpec((_ROWS, _C), lambda i: (i, 0)),
        out_shape=jax.ShapeDtypeStruct((_N, _C), jnp.float32),
        compiler_params=pltpu.CompilerParams(
            dimension_semantics=("arbitrary",),
        ),
    )(x3)


# speedup vs baseline: 2.0988x; 1.0178x over previous
import jax
import jax.numpy as jnp
from jax.experimental import pallas as pl
from jax.experimental.pallas import tpu as pltpu

_N = 16384
_C = 1000
_ROWS = 2048


def _onehot_block(x_ref, o_ref):
    xv = x_ref[0].reshape(_ROWS, 1)
    col = jax.lax.broadcasted_iota(jnp.int32, (_ROWS, _C), 1)
    o_ref[...] = (xv == col).astype(jnp.float32)


def kernel(x):
    x3 = x.reshape(_N // _ROWS, 1, _ROWS).astype(jnp.int32)
    return pl.pallas_call(
        _onehot_block,
        grid=(_N // _ROWS,),
        in_specs=[pl.BlockSpec((1, 1, _ROWS), lambda i: (i, 0, 0))],
        out_specs=pl.BlockSpec((_ROWS, _C), lambda i: (i, 0)),
        out_shape=jax.ShapeDtypeStruct((_N, _C), jnp.float32),
        compiler_params=pltpu.CompilerParams(
            dimension_semantics=("arbitrary",),
        ),
    )(x3)
